# baseline (speedup 1.0000x reference)
import jax
import jax.numpy as jnp
from jax.experimental import pallas as pl

_SEQ_BLOCK = 256


def _add_kernel(x_ref, e_ref, o_ref):
    o_ref[...] = x_ref[...] + e_ref[...][None, :, :]


def kernel(x, embeddings):
    b, t, d = x.shape
    emb = embeddings[:t]
    grid = (t // _SEQ_BLOCK,)
    return pl.pallas_call(
        _add_kernel,
        grid=grid,
        in_specs=[
            pl.BlockSpec((b, _SEQ_BLOCK, d), lambda i: (0, i, 0)),
            pl.BlockSpec((_SEQ_BLOCK, d), lambda i: (i, 0)),
        ],
        out_specs=pl.BlockSpec((b, _SEQ_BLOCK, d), lambda i: (0, i, 0)),
        out_shape=jax.ShapeDtypeStruct(x.shape, x.dtype),
    )(x, emb)


# speedup vs baseline: 1.0017x; 1.0017x over previous
import jax
import jax.numpy as jnp
from jax.experimental import pallas as pl

_SEQ_BLOCK = 512


def _add_kernel(x_ref, e_ref, o_ref):
    o_ref[...] = x_ref[...] + e_ref[...][None, :, :]


def kernel(x, embeddings):
    b, t, d = x.shape
    emb = embeddings[:t]
    grid = (t // _SEQ_BLOCK,)
    return pl.pallas_call(
        _add_kernel,
        grid=grid,
        in_specs=[
            pl.BlockSpec((b, _SEQ_BLOCK, d), lambda i: (0, i, 0)),
            pl.BlockSpec((_SEQ_BLOCK, d), lambda i: (i, 0)),
        ],
        out_specs=pl.BlockSpec((b, _SEQ_BLOCK, d), lambda i: (0, i, 0)),
        out_shape=jax.ShapeDtypeStruct(x.shape, x.dtype),
    )(x, emb)
